# baseline (speedup 1.0000x reference)
import functools
import math

import jax
import jax.numpy as jnp
from jax import lax
from jax.experimental import pallas as pl
from jax.experimental.pallas import tpu as pltpu
from jax.experimental.pallas import tpu_sc as plsc

D_MODEL = 128
LANES = 16
CHUNK = 128
NUM_CORES = 2
NUM_SUBCORES = 16
NUM_WORKERS = NUM_CORES * NUM_SUBCORES


def _body(x_hbm, idx_hbm, msk_hbm, pe_hbm, out_hbm,
          idx_v, msk_v, eff_v, x_v, rows_v, gsem):
    n_tokens = idx_hbm.shape[0]
    per_w = n_tokens // NUM_WORKERS
    n_chunks = per_w // CHUNK
    scale = math.sqrt(float(D_MODEL))
    pad = pe_hbm.shape[0] - 1

    wid = lax.axis_index("s") * NUM_CORES + lax.axis_index("c")
    base_w = wid * per_w

    def chunk_body(c, carry):
        base = base_w + c * CHUNK
        pltpu.sync_copy(idx_hbm.at[pl.ds(base, CHUNK)], idx_v)
        pltpu.sync_copy(msk_hbm.at[pl.ds(base, CHUNK)], msk_v)
        for j in range(CHUNK // LANES):
            sl = pl.ds(j * LANES, LANES)
            i = idx_v[sl]
            m = msk_v[sl]
            eff_v[sl] = jnp.where(m != 0, pad, jnp.minimum(i, pad))
        gcopy = pltpu.async_copy(pe_hbm.at[eff_v], rows_v, gsem)
        pltpu.sync_copy(x_hbm.at[pl.ds(base, CHUNK), :], x_v)
        gcopy.wait()

        def fma_body(t, carry2):
            for j in range(D_MODEL // LANES):
                sl = pl.ds(j * LANES, LANES)
                x_v[t, sl] = x_v[t, sl] * scale + rows_v[t, sl]
            return carry2

        lax.fori_loop(0, CHUNK, fma_body, 0)
        pltpu.sync_copy(x_v, out_hbm.at[pl.ds(base, CHUNK), :])
        return carry

    lax.fori_loop(0, n_chunks, chunk_body, 0)


def kernel(x, mask, indices, pe):
    b, s, d = x.shape
    n = b * s
    x2 = x.reshape(n, d)
    idx = indices.reshape(n).astype(jnp.int32)
    msk = mask.reshape(n).astype(jnp.int32)
    pe_eff = pe.at[pe.shape[0] - 1].set(0.0)

    mesh = plsc.VectorSubcoreMesh(core_axis_name="c", subcore_axis_name="s")
    run = functools.partial(
        pl.kernel,
        mesh=mesh,
        out_type=jax.ShapeDtypeStruct((n, d), jnp.float32),
        scratch_types=[
            pltpu.VMEM((CHUNK,), jnp.int32),
            pltpu.VMEM((CHUNK,), jnp.int32),
            pltpu.VMEM((CHUNK,), jnp.int32),
            pltpu.VMEM((CHUNK, D_MODEL), jnp.float32),
            pltpu.VMEM((CHUNK, D_MODEL), jnp.float32),
            pltpu.SemaphoreType.DMA,
        ],
    )(_body)
    out = run(x2, idx, msk, pe_eff)
    return out.reshape(b, s, d)
